# Initial kernel scaffold; baseline (speedup 1.0000x reference)
import jax
import jax.numpy as jnp
from jax.experimental import pallas as pl


def kernel(feats_node, edge_index, feats_graph, W1, b1, W2, b2, W3, b3, lw1, lb1, lw2, lb2, lw3, lb3):
    raise NotImplementedError("write your pallas kernel here")


# speedup vs baseline: 4.8496x; 4.8496x over previous
import functools

import jax
import jax.numpy as jnp
from jax import lax
from jax.experimental import pallas as pl
from jax.experimental.pallas import tpu as pltpu
from jax.experimental.pallas import tpu_sc as plsc

N = 10000
E = 320000
D = 128
NP = 10240
NC = 2
NS = 16
NW = NC * NS
EPW = E // NW
CHUNK = 80
NCHUNK = EPW // CHUNK
RPT = NP // NS
ZR = 128

_mesh = plsc.VectorSubcoreMesh(core_axis_name="c", subcore_axis_name="s")


def _zero_vmem_2d(ref, rows, cols):
    def body(i, carry):
        for k in range(cols // 16):
            ref[i, pl.ds(k * 16, 16)] = jnp.zeros((16,), jnp.float32)
        return carry
    lax.fori_loop(0, rows, body, 0)


@functools.partial(
    pl.kernel,
    out_type=(
        jax.ShapeDtypeStruct((NC, NP), jnp.float32),
        jax.ShapeDtypeStruct((NC, NP), jnp.float32),
    ),
    mesh=_mesh,
    scratch_types=[
        pltpu.VMEM((CHUNK,), jnp.int32),
        pltpu.VMEM((CHUNK,), jnp.int32),
        pltpu.VMEM((CHUNK,), jnp.float32),
        pltpu.VMEM_SHARED((NP,), jnp.float32),
        pltpu.VMEM_SHARED((NP,), jnp.float32),
    ],
)
def _sc_degrees(src_hbm, dst_hbm, ones_hbm, zcol_hbm, dego_out, degi_out,
                idx_src, idx_dst, ones, dego_sh, degi_sh):
    c = lax.axis_index("c")
    s = lax.axis_index("s")
    w = s * NC + c

    pltpu.sync_copy(ones_hbm, ones)
    pltpu.sync_copy(zcol_hbm, dego_sh.at[pl.ds(s * RPT, RPT)])
    pltpu.sync_copy(zcol_hbm, degi_sh.at[pl.ds(s * RPT, RPT)])
    plsc.subcore_barrier()

    base = w * EPW

    def chunk_body(j, carry):
        off = base + j * CHUNK
        pltpu.sync_copy(src_hbm.at[pl.ds(off, CHUNK)], idx_src)
        pltpu.sync_copy(dst_hbm.at[pl.ds(off, CHUNK)], idx_dst)
        pltpu.sync_copy(ones, dego_sh.at[idx_src], add=True)
        pltpu.sync_copy(ones, degi_sh.at[idx_dst], add=True)
        return carry
    lax.fori_loop(0, NCHUNK, chunk_body, 0)
    plsc.subcore_barrier()

    pltpu.sync_copy(dego_sh.at[pl.ds(s * RPT, RPT)],
                    dego_out.at[c, pl.ds(s * RPT, RPT)])
    pltpu.sync_copy(degi_sh.at[pl.ds(s * RPT, RPT)],
                    degi_out.at[c, pl.ds(s * RPT, RPT)])


def _make_spmm(compute_u):
    out_type = [jax.ShapeDtypeStruct((NC, NP, D), jnp.float32)]
    scratch = [
        pltpu.VMEM((CHUNK,), jnp.int32),
        pltpu.VMEM((CHUNK,), jnp.int32),
        pltpu.VMEM((CHUNK, D), jnp.float32),
        pltpu.VMEM((ZR, D), jnp.float32),
        pltpu.VMEM_SHARED((NP, D), jnp.float32),
        pltpu.SemaphoreType.DMA,
    ]
    if compute_u:
        out_type.append(jax.ShapeDtypeStruct((NC, NP), jnp.float32))
        scratch += [
            pltpu.VMEM((CHUNK,), jnp.float32),
            pltpu.VMEM_SHARED((NP,), jnp.float32),
            pltpu.SemaphoreType.DMA,
        ]

    def body(h_hbm, src_hbm, dst_hbm, nd_hbm, zcol_hbm, *rest):
        if compute_u:
            (agg_out, u_out, idx_src, idx_dst, rows, zbuf, agg_sh, sem,
             vals, u_sh, sem2) = rest
        else:
            agg_out, idx_src, idx_dst, rows, zbuf, agg_sh, sem = rest
        c = lax.axis_index("c")
        s = lax.axis_index("s")
        w = s * NC + c

        _zero_vmem_2d(zbuf, ZR, D)
        for t in range(RPT // ZR):
            pltpu.sync_copy(zbuf, agg_sh.at[pl.ds(s * RPT + t * ZR, ZR)])
        if compute_u:
            pltpu.sync_copy(zcol_hbm, u_sh.at[pl.ds(s * RPT, RPT)])
        plsc.subcore_barrier()

        base = w * EPW

        def chunk_body(j, carry):
            off = base + j * CHUNK
            pltpu.sync_copy(src_hbm.at[pl.ds(off, CHUNK)], idx_src)
            pltpu.sync_copy(dst_hbm.at[pl.ds(off, CHUNK)], idx_dst)
            pltpu.async_copy(h_hbm.at[idx_src], rows, sem).wait()
            pltpu.sync_copy(rows, agg_sh.at[idx_dst], add=True)
            if compute_u:
                pltpu.async_copy(nd_hbm.at[idx_dst], vals, sem2).wait()
                pltpu.sync_copy(vals, u_sh.at[idx_src], add=True)
            return carry
        lax.fori_loop(0, NCHUNK, chunk_body, 0)
        plsc.subcore_barrier()

        for t in range(RPT // ZR):
            pltpu.sync_copy(agg_sh.at[pl.ds(s * RPT + t * ZR, ZR)],
                            agg_out.at[c, pl.ds(s * RPT + t * ZR, ZR)])
        if compute_u:
            pltpu.sync_copy(u_sh.at[pl.ds(s * RPT, RPT)],
                            u_out.at[c, pl.ds(s * RPT, RPT)])

    out_t = tuple(out_type) if compute_u else out_type[0]
    return pl.kernel(body, out_type=out_t, mesh=_mesh,
                     scratch_types=scratch)


_sc_spmm_u = _make_spmm(True)
_sc_spmm = _make_spmm(False)


_BR = 1280
_GRID = NP // _BR


def _tc_norms_h1_body(x0_ref, w1_ref, dego_ref, degi_ref,
                      h1_ref, ns_ref, nd_ref):
    do_ = dego_ref[0] + dego_ref[1]
    di = degi_ref[0] + degi_ref[1]
    ns = lax.rsqrt(jnp.maximum(do_, 1.0))
    nd = lax.rsqrt(jnp.maximum(di, 1.0))
    ns_ref[...] = ns
    nd_ref[...] = nd
    h1_ref[...] = jnp.dot(x0_ref[...], w1_ref[...],
                          preferred_element_type=jnp.float32) * ns


def _tc_norms_h1(x0, w1, dego, degi):
    return pl.pallas_call(
        _tc_norms_h1_body,
        grid=(_GRID,),
        in_specs=[
            pl.BlockSpec((_BR, D), lambda i: (i, 0)),
            pl.BlockSpec((D, D), lambda i: (0, 0)),
            pl.BlockSpec((NC, _BR, 1), lambda i: (0, i, 0)),
            pl.BlockSpec((NC, _BR, 1), lambda i: (0, i, 0)),
        ],
        out_specs=[
            pl.BlockSpec((_BR, D), lambda i: (i, 0)),
            pl.BlockSpec((_BR, 1), lambda i: (i, 0)),
            pl.BlockSpec((_BR, 1), lambda i: (i, 0)),
        ],
        out_shape=[
            jax.ShapeDtypeStruct((NP, D), jnp.float32),
            jax.ShapeDtypeStruct((NP, 1), jnp.float32),
            jax.ShapeDtypeStruct((NP, 1), jnp.float32),
        ],
    )(x0, w1, dego, degi)


def _tc_layer_body(agg_ref, nd_ref, b_ref, w_next_ref, ns_ref, u_ref,
                   h_ref, wvec_ref):
    a = agg_ref[0] + agg_ref[1]
    x = jnp.maximum(a * nd_ref[...] + b_ref[...], 0.0)
    h_ref[...] = jnp.dot(x, w_next_ref[...],
                         preferred_element_type=jnp.float32) * ns_ref[...]
    wvec_ref[...] = ns_ref[...] * (u_ref[0] + u_ref[1])


def _tc_layer(agg, nd, b, w_next, ns, u):
    return pl.pallas_call(
        _tc_layer_body,
        grid=(_GRID,),
        in_specs=[
            pl.BlockSpec((NC, _BR, D), lambda i: (0, i, 0)),
            pl.BlockSpec((_BR, 1), lambda i: (i, 0)),
            pl.BlockSpec((1, D), lambda i: (0, 0)),
            pl.BlockSpec((D, D), lambda i: (0, 0)),
            pl.BlockSpec((_BR, 1), lambda i: (i, 0)),
            pl.BlockSpec((NC, _BR, 1), lambda i: (0, i, 0)),
        ],
        out_specs=[
            pl.BlockSpec((_BR, D), lambda i: (i, 0)),
            pl.BlockSpec((_BR, 1), lambda i: (i, 0)),
        ],
        out_shape=[
            jax.ShapeDtypeStruct((NP, D), jnp.float32),
            jax.ShapeDtypeStruct((NP, 1), jnp.float32),
        ],
    )(agg, nd, b, w_next, ns, u)


def _tc_reduce_body(agg_ref, nd_ref, b_ref, wvec_ref, r_ref):
    i = pl.program_id(0)
    a = agg_ref[0] + agg_ref[1]
    x2 = jnp.maximum(a * nd_ref[...] + b_ref[...], 0.0)
    partial = jnp.sum(x2 * wvec_ref[...], axis=0, keepdims=True)

    @pl.when(i == 0)
    def _():
        r_ref[...] = jnp.zeros_like(r_ref)
    r_ref[...] += partial


def _tc_reduce(agg, nd, b, wvec):
    return pl.pallas_call(
        _tc_reduce_body,
        grid=(_GRID,),
        in_specs=[
            pl.BlockSpec((NC, _BR, D), lambda i: (0, i, 0)),
            pl.BlockSpec((_BR, 1), lambda i: (i, 0)),
            pl.BlockSpec((1, D), lambda i: (0, 0)),
            pl.BlockSpec((_BR, 1), lambda i: (i, 0)),
        ],
        out_specs=pl.BlockSpec((1, D), lambda i: (0, 0)),
        out_shape=jax.ShapeDtypeStruct((1, D), jnp.float32),
    )(agg, nd, b, wvec)


def _tc_head_body(r_ref, w3_ref, b3_ref, fg_ref, lw1a_ref, lw1b_ref, lb1_ref,
                  lw2_ref, lb2_ref, lw3_ref, lb3_ref, out_ref):
    y = jnp.dot(r_ref[...], w3_ref[...],
                preferred_element_type=jnp.float32) * (1.0 / N) + b3_ref[...]
    t = (jnp.dot(y, lw1a_ref[...], preferred_element_type=jnp.float32)
         + jnp.dot(fg_ref[...], lw1b_ref[...],
                   preferred_element_type=jnp.float32) + lb1_ref[...])
    t = jnp.maximum(t, 0.0)
    t = jnp.maximum(jnp.dot(t, lw2_ref[...],
                            preferred_element_type=jnp.float32)
                    + lb2_ref[...], 0.0)
    out_ref[...] = jnp.dot(t, lw3_ref[...],
                           preferred_element_type=jnp.float32) + lb3_ref[...]


def _tc_head(r, w3, b3, fg, lw1a, lw1b, lb1, lw2, lb2, lw3, lb3):
    return pl.pallas_call(
        _tc_head_body,
        out_shape=jax.ShapeDtypeStruct((1, 1), jnp.float32),
    )(r, w3, b3, fg, lw1a, lw1b, lb1, lw2, lb2, lw3, lb3)


def kernel(feats_node, edge_index, feats_graph, W1, b1, W2, b2, W3, b3,
           lw1, lb1, lw2, lb2, lw3, lb3):
    src = edge_index[0].astype(jnp.int32)
    dst = edge_index[1].astype(jnp.int32)
    x0 = jnp.pad(feats_node, ((0, NP - N), (0, 0)))
    ones_col = jnp.ones((CHUNK,), jnp.float32)
    zcol = jnp.zeros((RPT,), jnp.float32)

    dego, degi = _sc_degrees(src, dst, ones_col, zcol)
    h1, ns, nd = _tc_norms_h1(x0, W1, dego.reshape(NC, NP, 1),
                              degi.reshape(NC, NP, 1))
    nd_flat = nd.reshape(NP)
    agg1, u = _sc_spmm_u(h1, src, dst, nd_flat, zcol)
    h2, wvec = _tc_layer(agg1, nd, b1.reshape(1, D), W2, ns,
                         u.reshape(NC, NP, 1))
    agg2 = _sc_spmm(h2, src, dst, nd_flat, zcol)
    r = _tc_reduce(agg2, nd, b2.reshape(1, D), wvec)

    fg = jnp.pad(feats_graph, ((0, 0), (0, 5)))
    lw1a = lw1[:D]
    lw1b = jnp.pad(lw1[D:], ((0, 5), (0, 0)))
    out = _tc_head(r, W3, b3.reshape(1, D), fg, lw1a, lw1b,
                   lb1.reshape(1, -1), lw2, lb2.reshape(1, -1),
                   lw3, lb3.reshape(1, 1))
    return out.reshape(-1)
